# baseline (speedup 1.0000x reference)
import functools

import jax
import jax.numpy as jnp
from jax.experimental import pallas as pl
from jax.experimental.pallas import tpu as pltpu

_NUM_E = 8192
_DIM = 256
_CC = 1.25

_BI = 4096
_BJ = 512


def _vq_loss_kernel(x_hbm, w_ref, out_ref, wbf_ref, w2_ref, xbuf_ref, sem,
                    *, scale, n_i):
    def _fetch(b, buf):
        return pltpu.make_async_copy(
            x_hbm.at[pl.ds(b * _BI, _BI), :], xbuf_ref.at[buf], sem.at[buf])

    _fetch(0, 0).start()

    wt = w_ref[...].T
    wbf_ref[...] = wt.astype(jnp.float8_e4m3fn)
    w2_ref[...] = jnp.sum(wt * wt, axis=0,
                          keepdims=True).astype(jnp.bfloat16)

    total = jnp.zeros((), jnp.float32)
    for b in range(n_i):
        cur = b % 2
        if b + 1 < n_i:
            _fetch(b + 1, 1 - cur).start()
        _fetch(b, cur).wait()
        x = xbuf_ref[cur]
        xs = (-2.0 * x).astype(jnp.float8_e4m3fn)

        m = jnp.full((_BI, 128), jnp.inf, dtype=jnp.bfloat16)
        for k in range(_NUM_E // _BJ):
            sl = slice(k * _BJ, (k + 1) * _BJ)
            wb = wbf_ref[:, sl]
            acc = jnp.dot(xs, wb, preferred_element_type=jnp.float32)
            d = acc.astype(jnp.bfloat16) + w2_ref[0, sl][None, :]
            for t in range(_BJ // 128):
                m = jnp.minimum(m, d[:, t * 128:(t + 1) * 128])

        row_min = jnp.min(m.astype(jnp.float32), axis=1)
        x2 = jnp.sum(x * x, axis=1)
        total = total + jnp.sum(row_min) + jnp.sum(x2)

    out_ref[...] = (total * scale).reshape(1, 1)


def kernel(inputs, weight):
    flat = inputs.reshape(-1, _DIM)
    n_rows = flat.shape[0]
    n_i = n_rows // _BI
    scale = (1.0 + _CC) / float(inputs.size)
    out = pl.pallas_call(
        functools.partial(_vq_loss_kernel, scale=scale, n_i=n_i),
        in_specs=[
            pl.BlockSpec(memory_space=pl.ANY),
            pl.BlockSpec((_NUM_E, _DIM), lambda: (0, 0)),
        ],
        out_specs=pl.BlockSpec((1, 1), lambda: (0, 0)),
        out_shape=jax.ShapeDtypeStruct((1, 1), jnp.float32),
        scratch_shapes=[
            pltpu.VMEM((_DIM, _NUM_E), jnp.float8_e4m3fn),
            pltpu.VMEM((1, _NUM_E), jnp.bfloat16),
            pltpu.VMEM((2, _BI, _DIM), jnp.float32),
            pltpu.SemaphoreType.DMA((2,)),
        ],
    )(flat, weight)
    return out[0, 0]


# speedup vs baseline: 1.0133x; 1.0133x over previous
import functools

import jax
import jax.numpy as jnp
from jax.experimental import pallas as pl
from jax.experimental.pallas import tpu as pltpu

_NUM_E = 8192
_DIM = 256
_CC = 1.25

_BI = 4096
_BJ = 512


def _vq_loss_kernel(x_ref, w_ref, out_ref, wbf_ref, w2_ref, *, scale):
    i = pl.program_id(0)

    @pl.when(i == 0)
    def _():
        wt = w_ref[...].T
        wbf_ref[...] = wt.astype(jnp.float8_e4m3fn)
        w2_ref[...] = jnp.sum(wt * wt, axis=0,
                              keepdims=True).astype(jnp.bfloat16)

    x = x_ref[...]
    xs = (-2.0 * x).astype(jnp.float8_e4m3fn)

    m = jnp.full((_BI, 128), jnp.inf, dtype=jnp.bfloat16)
    for k in range(_NUM_E // _BJ):
        sl = slice(k * _BJ, (k + 1) * _BJ)
        wb = wbf_ref[:, sl]
        acc = jnp.dot(xs, wb,
                      preferred_element_type=jnp.float32)
        d = acc.astype(jnp.bfloat16) + w2_ref[0, sl][None, :]
        for t in range(_BJ // 128):
            m = jnp.minimum(m, d[:, t * 128:(t + 1) * 128])

    row_min = jnp.min(m.astype(jnp.float32), axis=1)
    x2 = jnp.sum(x * x, axis=1)
    partial = ((jnp.sum(row_min) + jnp.sum(x2)) * scale).reshape(1, 1)

    @pl.when(i == 0)
    def _():
        out_ref[...] = jnp.zeros((1, 1), jnp.float32)
    out_ref[...] += partial


def kernel(inputs, weight):
    flat = inputs.reshape(-1, _DIM)
    n_rows = flat.shape[0]
    scale = (1.0 + _CC) / float(inputs.size)
    out = pl.pallas_call(
        functools.partial(_vq_loss_kernel, scale=scale),
        grid=(n_rows // _BI,),
        in_specs=[
            pl.BlockSpec((_BI, _DIM), lambda i: (i, 0)),
            pl.BlockSpec((_NUM_E, _DIM), lambda i: (0, 0)),
        ],
        out_specs=pl.BlockSpec((1, 1), lambda i: (0, 0)),
        out_shape=jax.ShapeDtypeStruct((1, 1), jnp.float32),
        scratch_shapes=[
            pltpu.VMEM((_DIM, _NUM_E), jnp.float8_e4m3fn),
            pltpu.VMEM((1, _NUM_E), jnp.bfloat16),
        ],
    )(flat, weight)
    return out[0, 0]
